# Initial kernel scaffold; baseline (speedup 1.0000x reference)
import functools

import jax
import jax.numpy as jnp
from jax import lax
from jax.experimental import pallas as pl
from jax.experimental.pallas import tpu as pltpu
from jax.experimental.pallas import tpu_sc as plsc

N = 10000
E = 320000
D = 128
H = 128

NC = 2
NS = 16
NW = NC * NS
K = 128
NCHUNK = E // K
JMAX = (NCHUNK + NW - 1) // NW
ROWS_PER_TILE = N // NS
ZR = 125
SEG = 2000


def _k1a_body(x_ref, w1_ref, as_ref, ad_ref, h_ref, hs_ref, hd_ref):
    h = jnp.dot(x_ref[...], w1_ref[...], preferred_element_type=jnp.float32)
    h_ref[...] = h
    hs_ref[...] = jnp.dot(h, as_ref[...], preferred_element_type=jnp.float32)
    hd_ref[...] = jnp.dot(h, ad_ref[...], preferred_element_type=jnp.float32)


def _node_matmuls(x, W1, a_s, a_d):
    B = 2000
    return pl.pallas_call(
        _k1a_body,
        grid=(N // B,),
        in_specs=[
            pl.BlockSpec((B, D), lambda i: (i, 0)),
            pl.BlockSpec((D, H), lambda i: (0, 0)),
            pl.BlockSpec((H, 1), lambda i: (0, 0)),
            pl.BlockSpec((H, 1), lambda i: (0, 0)),
        ],
        out_specs=[
            pl.BlockSpec((B, H), lambda i: (i, 0)),
            pl.BlockSpec((B, 1), lambda i: (i, 0)),
            pl.BlockSpec((B, 1), lambda i: (i, 0)),
        ],
        out_shape=[
            jax.ShapeDtypeStruct((N, H), jnp.float32),
            jax.ShapeDtypeStruct((N, 1), jnp.float32),
            jax.ShapeDtypeStruct((N, 1), jnp.float32),
        ],
    )(x, W1, a_s.reshape(H, 1), a_d.reshape(H, 1))


def _k1b_body(ea_ref, we1_ref, ae1_ref, we2_ref, ae2_ref, eav_ref):
    v1 = jnp.dot(we1_ref[...], ae1_ref[...], preferred_element_type=jnp.float32)
    v2 = jnp.dot(we2_ref[...], ae2_ref[...], preferred_element_type=jnp.float32)
    vmat = jnp.concatenate([v1, v2], axis=1)
    eav_ref[...] = jnp.dot(ea_ref[...], vmat, preferred_element_type=jnp.float32)


def _edge_scalars(edge_attr, We1, a_e1, We2, a_e2):
    B = 32000
    DE = edge_attr.shape[1]
    return pl.pallas_call(
        _k1b_body,
        grid=(E // B,),
        in_specs=[
            pl.BlockSpec((B, DE), lambda i: (i, 0)),
            pl.BlockSpec((DE, H), lambda i: (0, 0)),
            pl.BlockSpec((H, 1), lambda i: (0, 0)),
            pl.BlockSpec((DE, 1), lambda i: (0, 0)),
            pl.BlockSpec((1, 1), lambda i: (0, 0)),
        ],
        out_specs=pl.BlockSpec((B, 2), lambda i: (i, 0)),
        out_shape=jax.ShapeDtypeStruct((E, 2), jnp.float32),
    )(edge_attr, We1, a_e1.reshape(H, 1), We2, a_e2.reshape(1, 1))


_mesh = plsc.VectorSubcoreMesh(core_axis_name="c", subcore_axis_name="s",
                               num_cores=NC, num_subcores=NS)


@functools.partial(
    pl.kernel,
    out_type=[
        jax.ShapeDtypeStruct((NC, N, H), jnp.float32),
        jax.ShapeDtypeStruct((NC, N), jnp.float32),
        jax.ShapeDtypeStruct((NC, N), jnp.float32),
        jax.ShapeDtypeStruct((NC, N), jnp.float32),
        jax.ShapeDtypeStruct((NC, N), jnp.float32),
    ],
    mesh=_mesh,
    scratch_types=[
        pltpu.VMEM_SHARED((N, H), jnp.float32),
        pltpu.VMEM_SHARED((N,), jnp.float32),
        pltpu.VMEM_SHARED((N,), jnp.float32),
        pltpu.VMEM_SHARED((N,), jnp.float32),
        pltpu.VMEM_SHARED((N,), jnp.float32),
        pltpu.VMEM((K,), jnp.int32),
        pltpu.VMEM((1, K), jnp.int32),
        pltpu.VMEM((K,), jnp.float32),
        pltpu.VMEM((K,), jnp.float32),
        pltpu.VMEM((K,), jnp.float32),
        pltpu.VMEM((K,), jnp.float32),
        pltpu.VMEM((K, H), jnp.float32),
        pltpu.VMEM((N,), jnp.float32),
        pltpu.VMEM((N,), jnp.float32),
        pltpu.VMEM((ZR, H), jnp.float32),
        pltpu.VMEM((SEG,), jnp.float32),
        pltpu.SemaphoreType.DMA,
    ],
)
def _edge_pass1(src_hbm, dst_hbm, eav1_hbm, eav2_hbm, hs_hbm, hd_hbm, h_hbm,
                outp_hbm, cnt_hbm, sev1_hbm, sev2_hbm, den_hbm,
                out_sh, cnt_sh, sev1_sh, sev2_sh, den_sh,
                src_i, dst2d, eav1_v, eav2_v, p_v, ones_v, rows_v,
                hs_v, hd_v, zrow_v, z1d_v, sem):
    cid = lax.axis_index("c")
    sid = lax.axis_index("s")
    wid = sid * NC + cid

    zero16 = jnp.zeros((16,), jnp.float32)
    one16 = jnp.ones((16,), jnp.float32)
    for i in range(K // 16):
        ones_v[pl.ds(i * 16, 16)] = one16

    def _zrow(r, c):
        for j in range(H // 16):
            zrow_v[r, pl.ds(j * 16, 16)] = zero16
        return c
    lax.fori_loop(0, ZR, _zrow, 0)

    def _z1d(r, c):
        z1d_v[pl.ds(r * 16, 16)] = zero16
        return c
    lax.fori_loop(0, SEG // 16, _z1d, 0)

    pltpu.sync_copy(hs_hbm, hs_v)
    pltpu.sync_copy(hd_hbm, hd_v)

    for t in range(ROWS_PER_TILE // ZR):
        pltpu.sync_copy(zrow_v, out_sh.at[pl.ds(sid * ROWS_PER_TILE + t * ZR, ZR)])

    @pl.when(sid < N // SEG)
    def _():
        pltpu.sync_copy(z1d_v, cnt_sh.at[pl.ds(sid * SEG, SEG)])
        pltpu.sync_copy(z1d_v, sev1_sh.at[pl.ds(sid * SEG, SEG)])
        pltpu.sync_copy(z1d_v, sev2_sh.at[pl.ds(sid * SEG, SEG)])
        pltpu.sync_copy(z1d_v, den_sh.at[pl.ds(sid * SEG, SEG)])

    plsc.subcore_barrier()

    def _chunk(j, c):
        chunk = wid + NW * j

        @pl.when(chunk < NCHUNK)
        def _():
            base = chunk * K
            pltpu.sync_copy(src_hbm.at[pl.ds(base, K)], src_i)
            pltpu.sync_copy(dst_hbm.at[pl.ds(base, K)], dst2d.at[0])
            pltpu.sync_copy(eav1_hbm.at[pl.ds(base, K)], eav1_v)
            pltpu.sync_copy(eav2_hbm.at[pl.ds(base, K)], eav2_v)
            gath = pltpu.async_copy(h_hbm.at[src_i], rows_v, sem)
            for i in range(K // 16):
                sv = plsc.load_gather(hs_v, [src_i[pl.ds(i * 16, 16)]])
                dv = plsc.load_gather(hd_v, [dst2d[0, pl.ds(i * 16, 16)]])
                a = sv + dv + eav1_v[pl.ds(i * 16, 16)]
                a = jnp.maximum(a, 0.2 * a)
                p_v[pl.ds(i * 16, 16)] = jnp.exp(a)
            gath.wait()

            def _scale(r, cc):
                w = jnp.full((16,), p_v[r])
                for col in range(H // 16):
                    rows_v[r, pl.ds(col * 16, 16)] = (
                        rows_v[r, pl.ds(col * 16, 16)] * w)
                return cc
            lax.fori_loop(0, K, _scale, 0)

            pltpu.sync_copy(ones_v, cnt_sh.at[dst2d.at[0]], add=True)
            pltpu.sync_copy(eav1_v, sev1_sh.at[dst2d.at[0]], add=True)
            pltpu.sync_copy(eav2_v, sev2_sh.at[dst2d.at[0]], add=True)
            pltpu.sync_copy(p_v, den_sh.at[dst2d.at[0]], add=True)
            pltpu.sync_copy(rows_v, out_sh.at[dst2d.at[0]], add=True)
        return c
    lax.fori_loop(0, JMAX, _chunk, 0)

    plsc.subcore_barrier()

    pltpu.sync_copy(out_sh.at[pl.ds(sid * ROWS_PER_TILE, ROWS_PER_TILE)],
                    outp_hbm.at[cid, pl.ds(sid * ROWS_PER_TILE, ROWS_PER_TILE)])

    @pl.when(sid < N // SEG)
    def _():
        pltpu.sync_copy(cnt_sh.at[pl.ds(sid * SEG, SEG)],
                        cnt_hbm.at[cid, pl.ds(sid * SEG, SEG)])
        pltpu.sync_copy(sev1_sh.at[pl.ds(sid * SEG, SEG)],
                        sev1_hbm.at[cid, pl.ds(sid * SEG, SEG)])
        pltpu.sync_copy(sev2_sh.at[pl.ds(sid * SEG, SEG)],
                        sev2_hbm.at[cid, pl.ds(sid * SEG, SEG)])
        pltpu.sync_copy(den_sh.at[pl.ds(sid * SEG, SEG)],
                        den_hbm.at[cid, pl.ds(sid * SEG, SEG)])


@functools.partial(
    pl.kernel,
    out_type=[
        jax.ShapeDtypeStruct((NC, N), jnp.float32),
        jax.ShapeDtypeStruct((NC, N), jnp.float32),
    ],
    mesh=_mesh,
    scratch_types=[
        pltpu.VMEM_SHARED((N,), jnp.float32),
        pltpu.VMEM_SHARED((N,), jnp.float32),
        pltpu.VMEM((K,), jnp.int32),
        pltpu.VMEM((1, K), jnp.int32),
        pltpu.VMEM((K,), jnp.float32),
        pltpu.VMEM((K,), jnp.float32),
        pltpu.VMEM((K,), jnp.float32),
        pltpu.VMEM((N,), jnp.float32),
        pltpu.VMEM((N,), jnp.float32),
        pltpu.VMEM((N,), jnp.float32),
        pltpu.VMEM((SEG,), jnp.float32),
    ],
)
def _edge_pass2(src_hbm, dst_hbm, eav2_hbm, g_hbm, gs_hbm, gd_hbm,
                den2_hbm, num2_hbm,
                den2_sh, num2_sh,
                src_i, dst2d, eav2_v, p2_v, pn_v, g_v, gs_v, gd_v, z1d_v):
    cid = lax.axis_index("c")
    sid = lax.axis_index("s")
    wid = sid * NC + cid

    zero16 = jnp.zeros((16,), jnp.float32)

    def _z1d(r, c):
        z1d_v[pl.ds(r * 16, 16)] = zero16
        return c
    lax.fori_loop(0, SEG // 16, _z1d, 0)

    pltpu.sync_copy(g_hbm, g_v)
    pltpu.sync_copy(gs_hbm, gs_v)
    pltpu.sync_copy(gd_hbm, gd_v)

    @pl.when(sid < N // SEG)
    def _():
        pltpu.sync_copy(z1d_v, den2_sh.at[pl.ds(sid * SEG, SEG)])
        pltpu.sync_copy(z1d_v, num2_sh.at[pl.ds(sid * SEG, SEG)])

    plsc.subcore_barrier()

    def _chunk(j, c):
        chunk = wid + NW * j

        @pl.when(chunk < NCHUNK)
        def _():
            base = chunk * K
            pltpu.sync_copy(src_hbm.at[pl.ds(base, K)], src_i)
            pltpu.sync_copy(dst_hbm.at[pl.ds(base, K)], dst2d.at[0])
            pltpu.sync_copy(eav2_hbm.at[pl.ds(base, K)], eav2_v)
            for i in range(K // 16):
                sl = pl.ds(i * 16, 16)
                gsv = plsc.load_gather(gs_v, [src_i[sl]])
                gdv = plsc.load_gather(gd_v, [dst2d[0, sl]])
                gv = plsc.load_gather(g_v, [src_i[sl]])
                a = gsv + gdv + eav2_v[sl]
                a = jnp.maximum(a, 0.2 * a)
                p = jnp.exp(a)
                p2_v[sl] = p
                pn_v[sl] = p * gv
            pltpu.sync_copy(p2_v, den2_sh.at[dst2d.at[0]], add=True)
            pltpu.sync_copy(pn_v, num2_sh.at[dst2d.at[0]], add=True)
        return c
    lax.fori_loop(0, JMAX, _chunk, 0)

    plsc.subcore_barrier()

    @pl.when(sid < N // SEG)
    def _():
        pltpu.sync_copy(den2_sh.at[pl.ds(sid * SEG, SEG)],
                        den2_hbm.at[cid, pl.ds(sid * SEG, SEG)])
        pltpu.sync_copy(num2_sh.at[pl.ds(sid * SEG, SEG)],
                        num2_hbm.at[cid, pl.ds(sid * SEG, SEG)])


def _k4a_body(hs_ref, hd_ref, cnt_ref, sev1_ref, sev2_ref, den_ref,
              p1l_ref, inv1_ref, lv2_ref):
    cnt = cnt_ref[...][0] + cnt_ref[...][1]
    sev1 = sev1_ref[...][0] + sev1_ref[...][1]
    sev2 = sev2_ref[...][0] + sev2_ref[...][1]
    den = den_ref[...][0] + den_ref[...][1]
    cmax = jnp.maximum(cnt, 1.0)
    a1 = hs_ref[...][:, 0] + hd_ref[...][:, 0] + sev1 / cmax
    a1 = jnp.maximum(a1, 0.2 * a1)
    p1l = jnp.exp(a1)
    p1l_ref[...] = p1l[:, None]
    inv1_ref[...] = (1.0 / (den + p1l + 1e-16))[:, None]
    lv2_ref[...] = (sev2 / cmax)[:, None]


def _node_scalars(hs, hd, cnt_p, sev1_p, sev2_p, den_p):
    return pl.pallas_call(
        _k4a_body,
        out_shape=[
            jax.ShapeDtypeStruct((N, 1), jnp.float32),
            jax.ShapeDtypeStruct((N, 1), jnp.float32),
            jax.ShapeDtypeStruct((N, 1), jnp.float32),
        ],
    )(hs, hd, cnt_p, sev1_p, sev2_p, den_p)


def _k4b_body(outp_ref, h_ref, p1l_ref, inv1_ref, lv2_ref, w2_ref, sc2_ref,
              b1_ref, g_ref, gs_ref, gd_ref, p2l_ref, p2lg_ref):
    op = outp_ref[...]
    p1l = p1l_ref[...]
    inv1 = inv1_ref[...]
    out1 = (op[0] + op[1] + p1l * h_ref[...]) * inv1 + b1_ref[...]
    h1 = jnp.where(out1 > 0, out1, jnp.expm1(out1))
    g = jnp.dot(h1, w2_ref[...], preferred_element_type=jnp.float32)
    as2 = sc2_ref[0, 0]
    ad2 = sc2_ref[0, 1]
    g_ref[...] = g
    gs_ref[...] = as2 * g
    gd_ref[...] = ad2 * g
    a2 = (as2 + ad2) * g + lv2_ref[...]
    a2 = jnp.maximum(a2, 0.2 * a2)
    p2l = jnp.exp(a2)
    p2l_ref[...] = p2l
    p2lg_ref[...] = p2l * g


def _combine1(outp, h, p1l, inv1, lv2, W2, sc2, b1):
    B = 2000
    return pl.pallas_call(
        _k4b_body,
        grid=(N // B,),
        in_specs=[
            pl.BlockSpec((NC, B, H), lambda i: (0, i, 0)),
            pl.BlockSpec((B, H), lambda i: (i, 0)),
            pl.BlockSpec((B, 1), lambda i: (i, 0)),
            pl.BlockSpec((B, 1), lambda i: (i, 0)),
            pl.BlockSpec((B, 1), lambda i: (i, 0)),
            pl.BlockSpec((H, 1), lambda i: (0, 0)),
            pl.BlockSpec(memory_space=pltpu.SMEM),
            pl.BlockSpec((1, H), lambda i: (0, 0)),
        ],
        out_specs=[pl.BlockSpec((B, 1), lambda i: (i, 0))] * 5,
        out_shape=[jax.ShapeDtypeStruct((N, 1), jnp.float32)] * 5,
    )(outp, h, p1l, inv1, lv2, W2, sc2, b1)


def _k6_body(den2_ref, num2_ref, p2l_ref, p2lg_ref, b2_ref, out_ref):
    den = den2_ref[...][0] + den2_ref[...][1] + p2l_ref[...][:, 0]
    num = num2_ref[...][0] + num2_ref[...][1] + p2lg_ref[...][:, 0]
    out_ref[...] = (num / (den + 1e-16))[:, None] + b2_ref[0, 0]


def _combine2(den2_p, num2_p, p2l, p2lg, b2):
    return pl.pallas_call(
        _k6_body,
        in_specs=[
            pl.BlockSpec(),
            pl.BlockSpec(),
            pl.BlockSpec(),
            pl.BlockSpec(),
            pl.BlockSpec(memory_space=pltpu.SMEM),
        ],
        out_shape=jax.ShapeDtypeStruct((N, 1), jnp.float32),
    )(den2_p, num2_p, p2l, p2lg, b2)


def kernel(x, edge_index, edge_attr, W1, a_src1, a_dst1, We1, a_edge1, b1,
           W2, a_src2, a_dst2, We2, a_edge2, b2):
    src = edge_index[0].astype(jnp.int32)
    dst = edge_index[1].astype(jnp.int32)

    h, hs, hd = _node_matmuls(x, W1, a_src1, a_dst1)
    eav = _edge_scalars(edge_attr, We1, a_edge1, We2, a_edge2)
    eav1 = jnp.ascontiguousarray(eav[:, 0])
    eav2 = jnp.ascontiguousarray(eav[:, 1])

    outp, cnt_p, sev1_p, sev2_p, den_p = _edge_pass1(
        src, dst, eav1, eav2, hs.reshape(N), hd.reshape(N), h)

    p1l, inv1, lv2 = _node_scalars(hs, hd, cnt_p, sev1_p, sev2_p, den_p)

    sc2 = jnp.stack([a_src2[0], a_dst2[0]]).reshape(1, 2)
    g, gs, gd, p2l, p2lg = _combine1(outp, h, p1l, inv1, lv2,
                                     W2, sc2, b1.reshape(1, H))

    den2_p, num2_p = _edge_pass2(src, dst, eav2, g.reshape(N),
                                 gs.reshape(N), gd.reshape(N))

    out = _combine2(den2_p, num2_p, p2l, p2lg, b2.reshape(1, 1))
    return out


# speedup vs baseline: 24.7205x; 24.7205x over previous
import functools

import jax
import jax.numpy as jnp
from jax import lax
from jax.experimental import pallas as pl
from jax.experimental.pallas import tpu as pltpu
from jax.experimental.pallas import tpu_sc as plsc

N = 10000
E = 320000
D = 128
H = 128

NC = 2
NS = 16
NW = NC * NS
K = 128
NCHUNK = E // K
JMAX = (NCHUNK + NW - 1) // NW
NP = 10240
RT = NP // NS
ZR = 128


def _k1a_body(x_ref, w1_ref, as_ref, ad_ref, h_ref, hs_ref, hd_ref):
    h = jnp.dot(x_ref[...], w1_ref[...], preferred_element_type=jnp.float32)
    h_ref[...] = h
    hs_ref[...] = jnp.dot(h, as_ref[...], preferred_element_type=jnp.float32)
    hd_ref[...] = jnp.dot(h, ad_ref[...], preferred_element_type=jnp.float32)


def _node_matmuls(xp, W1, a_s, a_d):
    B = 2048
    return pl.pallas_call(
        _k1a_body,
        grid=(NP // B,),
        in_specs=[
            pl.BlockSpec((B, D), lambda i: (i, 0)),
            pl.BlockSpec((D, H), lambda i: (0, 0)),
            pl.BlockSpec((H, 1), lambda i: (0, 0)),
            pl.BlockSpec((H, 1), lambda i: (0, 0)),
        ],
        out_specs=[
            pl.BlockSpec((B, H), lambda i: (i, 0)),
            pl.BlockSpec((B, 1), lambda i: (i, 0)),
            pl.BlockSpec((B, 1), lambda i: (i, 0)),
        ],
        out_shape=[
            jax.ShapeDtypeStruct((NP, H), jnp.float32),
            jax.ShapeDtypeStruct((NP, 1), jnp.float32),
            jax.ShapeDtypeStruct((NP, 1), jnp.float32),
        ],
    )(xp, W1, a_s.reshape(H, 1), a_d.reshape(H, 1))


def _k1b_body(ea_ref, we1_ref, ae1_ref, we2_ref, ae2_ref, eav_ref):
    v1 = jnp.dot(we1_ref[...], ae1_ref[...], preferred_element_type=jnp.float32)
    v2 = jnp.dot(we2_ref[...], ae2_ref[...], preferred_element_type=jnp.float32)
    vmat = jnp.concatenate([v1, v2], axis=1)
    eav_ref[...] = jnp.dot(ea_ref[...], vmat, preferred_element_type=jnp.float32)


def _edge_scalars(edge_attr, We1, a_e1, We2, a_e2):
    B = 8000
    DE = edge_attr.shape[1]
    return pl.pallas_call(
        _k1b_body,
        grid=(E // B,),
        in_specs=[
            pl.BlockSpec((B, DE), lambda i: (i, 0)),
            pl.BlockSpec((DE, H), lambda i: (0, 0)),
            pl.BlockSpec((H, 1), lambda i: (0, 0)),
            pl.BlockSpec((DE, 1), lambda i: (0, 0)),
            pl.BlockSpec((1, 1), lambda i: (0, 0)),
        ],
        out_specs=pl.BlockSpec((B, 2), lambda i: (i, 0)),
        out_shape=jax.ShapeDtypeStruct((E, 2), jnp.float32),
    )(edge_attr, We1, a_e1.reshape(H, 1), We2, a_e2.reshape(1, 1))


@functools.cache
def _get_edge_pass1():
    mesh = plsc.VectorSubcoreMesh(core_axis_name="c", subcore_axis_name="s",
                                  num_cores=NC, num_subcores=NS)
    return functools.partial(
        pl.kernel,
        out_type=[
            jax.ShapeDtypeStruct((NC, NP, H), jnp.float32),
            jax.ShapeDtypeStruct((NC * NP,), jnp.float32),
            jax.ShapeDtypeStruct((NC * NP,), jnp.float32),
            jax.ShapeDtypeStruct((NC * NP,), jnp.float32),
            jax.ShapeDtypeStruct((NC * NP,), jnp.float32),
        ],
        mesh=mesh,
        compiler_params=pltpu.CompilerParams(needs_layout_passes=False),
        scratch_types=[
            pltpu.VMEM_SHARED((NP, H), jnp.float32),
            pltpu.VMEM_SHARED((NP,), jnp.float32),
            pltpu.VMEM_SHARED((NP,), jnp.float32),
            pltpu.VMEM_SHARED((NP,), jnp.float32),
            pltpu.VMEM_SHARED((NP,), jnp.float32),
            pltpu.VMEM((K,), jnp.int32),
            pltpu.VMEM((1, K), jnp.int32),
            pltpu.VMEM((K,), jnp.float32),
            pltpu.VMEM((K,), jnp.float32),
            pltpu.VMEM((K,), jnp.float32),
            pltpu.VMEM((K,), jnp.float32),
            pltpu.VMEM((K, H), jnp.float32),
            pltpu.VMEM((NP,), jnp.float32),
            pltpu.VMEM((NP,), jnp.float32),
            pltpu.VMEM((RT,), jnp.float32),
            pltpu.SemaphoreType.DMA,
        ],
    )(_edge_pass1_body)


def _edge_pass1_body(src_hbm, dst_hbm, eav1_hbm, eav2_hbm, hs_hbm, hd_hbm, h_hbm,
                     outp_hbm, cnt_hbm, sev1_hbm, sev2_hbm, den_hbm,
                     out_sh, cnt_sh, sev1_sh, sev2_sh, den_sh,
                     src_i, dst2d, eav1_v, eav2_v, p_v, ones_v, rows_v,
                     hs_v, hd_v, z1d_v, sem):
    cid = lax.axis_index("c")
    sid = lax.axis_index("s")
    wid = sid * NC + cid

    zero16 = jnp.zeros((16,), jnp.float32)
    one16 = jnp.ones((16,), jnp.float32)
    for i in range(K // 16):
        ones_v[pl.ds(i * 16, 16)] = one16

    def _zrow(r, c):
        for j in range(H // 16):
            rows_v[r, pl.ds(j * 16, 16)] = zero16
        return c
    lax.fori_loop(0, ZR, _zrow, 0)

    def _z1d(r, c):
        z1d_v[pl.ds(r * 16, 16)] = zero16
        return c
    lax.fori_loop(0, RT // 16, _z1d, 0)

    pltpu.sync_copy(hs_hbm, hs_v)
    pltpu.sync_copy(hd_hbm, hd_v)

    for t in range(RT // ZR):
        pltpu.sync_copy(rows_v, out_sh.at[pl.ds(sid * RT + t * ZR, ZR)])
    pltpu.sync_copy(z1d_v, cnt_sh.at[pl.ds(sid * RT, RT)])
    pltpu.sync_copy(z1d_v, sev1_sh.at[pl.ds(sid * RT, RT)])
    pltpu.sync_copy(z1d_v, sev2_sh.at[pl.ds(sid * RT, RT)])
    pltpu.sync_copy(z1d_v, den_sh.at[pl.ds(sid * RT, RT)])

    plsc.subcore_barrier()

    def _chunk(j, c):
        chunk = wid + NW * j

        @pl.when(chunk < NCHUNK)
        def _():
            base = chunk * K
            pltpu.sync_copy(src_hbm.at[pl.ds(base, K)], src_i)
            pltpu.sync_copy(dst_hbm.at[pl.ds(base, K)], dst2d.at[0])
            pltpu.sync_copy(eav1_hbm.at[pl.ds(base, K)], eav1_v)
            pltpu.sync_copy(eav2_hbm.at[pl.ds(base, K)], eav2_v)
            gath = pltpu.async_copy(h_hbm.at[src_i], rows_v, sem)
            for i in range(K // 16):
                sv = plsc.load_gather(hs_v, [src_i[pl.ds(i * 16, 16)]])
                dv = plsc.load_gather(hd_v, [dst2d[0, pl.ds(i * 16, 16)]])
                a = sv + dv + eav1_v[pl.ds(i * 16, 16)]
                a = jnp.maximum(a, 0.2 * a)
                p_v[pl.ds(i * 16, 16)] = jnp.exp(a)
            gath.wait()

            def _scale(gidx, cc):
                w16 = p_v[pl.ds(gidx * 16, 16)]
                for l in range(16):
                    row = gidx * 16 + l
                    w = jnp.full((16,), w16[l])
                    for col in range(H // 16):
                        rows_v[row, pl.ds(col * 16, 16)] = (
                            rows_v[row, pl.ds(col * 16, 16)] * w)
                return cc
            lax.fori_loop(0, K // 16, _scale, 0)

            pltpu.sync_copy(ones_v, cnt_sh.at[dst2d.at[0]], add=True)
            pltpu.sync_copy(eav1_v, sev1_sh.at[dst2d.at[0]], add=True)
            pltpu.sync_copy(eav2_v, sev2_sh.at[dst2d.at[0]], add=True)
            pltpu.sync_copy(p_v, den_sh.at[dst2d.at[0]], add=True)
            pltpu.sync_copy(rows_v, out_sh.at[dst2d.at[0]], add=True)
        return c
    lax.fori_loop(0, JMAX, _chunk, 0)

    plsc.subcore_barrier()

    pltpu.sync_copy(out_sh.at[pl.ds(sid * RT, RT)],
                    outp_hbm.at[cid, pl.ds(sid * RT, RT)])
    flat = cid * NP + sid * RT
    pltpu.sync_copy(cnt_sh.at[pl.ds(sid * RT, RT)], cnt_hbm.at[pl.ds(flat, RT)])
    pltpu.sync_copy(sev1_sh.at[pl.ds(sid * RT, RT)], sev1_hbm.at[pl.ds(flat, RT)])
    pltpu.sync_copy(sev2_sh.at[pl.ds(sid * RT, RT)], sev2_hbm.at[pl.ds(flat, RT)])
    pltpu.sync_copy(den_sh.at[pl.ds(sid * RT, RT)], den_hbm.at[pl.ds(flat, RT)])


@functools.cache
def _get_edge_pass2():
    mesh = plsc.VectorSubcoreMesh(core_axis_name="c", subcore_axis_name="s",
                                  num_cores=NC, num_subcores=NS)
    return functools.partial(
        pl.kernel,
        out_type=[
            jax.ShapeDtypeStruct((NC * NP,), jnp.float32),
            jax.ShapeDtypeStruct((NC * NP,), jnp.float32),
        ],
        mesh=mesh,
        compiler_params=pltpu.CompilerParams(needs_layout_passes=False),
        scratch_types=[
            pltpu.VMEM_SHARED((NP,), jnp.float32),
            pltpu.VMEM_SHARED((NP,), jnp.float32),
            pltpu.VMEM((K,), jnp.int32),
            pltpu.VMEM((1, K), jnp.int32),
            pltpu.VMEM((K,), jnp.float32),
            pltpu.VMEM((K,), jnp.float32),
            pltpu.VMEM((K,), jnp.float32),
            pltpu.VMEM((NP,), jnp.float32),
            pltpu.VMEM((NP,), jnp.float32),
            pltpu.VMEM((NP,), jnp.float32),
            pltpu.VMEM((RT,), jnp.float32),
        ],
    )(_edge_pass2_body)


def _edge_pass2_body(src_hbm, dst_hbm, eav2_hbm, g_hbm, gs_hbm, gd_hbm,
                     den2_hbm, num2_hbm,
                     den2_sh, num2_sh,
                     src_i, dst2d, eav2_v, p2_v, pn_v, g_v, gs_v, gd_v, z1d_v):
    cid = lax.axis_index("c")
    sid = lax.axis_index("s")
    wid = sid * NC + cid

    zero16 = jnp.zeros((16,), jnp.float32)

    def _z1d(r, c):
        z1d_v[pl.ds(r * 16, 16)] = zero16
        return c
    lax.fori_loop(0, RT // 16, _z1d, 0)

    pltpu.sync_copy(g_hbm, g_v)
    pltpu.sync_copy(gs_hbm, gs_v)
    pltpu.sync_copy(gd_hbm, gd_v)

    pltpu.sync_copy(z1d_v, den2_sh.at[pl.ds(sid * RT, RT)])
    pltpu.sync_copy(z1d_v, num2_sh.at[pl.ds(sid * RT, RT)])

    plsc.subcore_barrier()

    def _chunk(j, c):
        chunk = wid + NW * j

        @pl.when(chunk < NCHUNK)
        def _():
            base = chunk * K
            pltpu.sync_copy(src_hbm.at[pl.ds(base, K)], src_i)
            pltpu.sync_copy(dst_hbm.at[pl.ds(base, K)], dst2d.at[0])
            pltpu.sync_copy(eav2_hbm.at[pl.ds(base, K)], eav2_v)
            for i in range(K // 16):
                sl = pl.ds(i * 16, 16)
                gsv = plsc.load_gather(gs_v, [src_i[sl]])
                gdv = plsc.load_gather(gd_v, [dst2d[0, sl]])
                gv = plsc.load_gather(g_v, [src_i[sl]])
                a = gsv + gdv + eav2_v[sl]
                a = jnp.maximum(a, 0.2 * a)
                p = jnp.exp(a)
                p2_v[sl] = p
                pn_v[sl] = p * gv
            pltpu.sync_copy(p2_v, den2_sh.at[dst2d.at[0]], add=True)
            pltpu.sync_copy(pn_v, num2_sh.at[dst2d.at[0]], add=True)
        return c
    lax.fori_loop(0, JMAX, _chunk, 0)

    plsc.subcore_barrier()

    flat = cid * NP + sid * RT
    pltpu.sync_copy(den2_sh.at[pl.ds(sid * RT, RT)], den2_hbm.at[pl.ds(flat, RT)])
    pltpu.sync_copy(num2_sh.at[pl.ds(sid * RT, RT)], num2_hbm.at[pl.ds(flat, RT)])


def _k4a_body(hs_ref, hd_ref, cnt_ref, sev1_ref, sev2_ref, den_ref,
              p1l_ref, inv1_ref, lv2_ref):
    cnt = cnt_ref[...][0] + cnt_ref[...][1]
    sev1 = sev1_ref[...][0] + sev1_ref[...][1]
    sev2 = sev2_ref[...][0] + sev2_ref[...][1]
    den = den_ref[...][0] + den_ref[...][1]
    cmax = jnp.maximum(cnt, 1.0)
    a1 = hs_ref[...][:, 0] + hd_ref[...][:, 0] + sev1 / cmax
    a1 = jnp.maximum(a1, 0.2 * a1)
    p1l = jnp.exp(a1)
    p1l_ref[...] = p1l[:, None]
    inv1_ref[...] = (1.0 / (den + p1l + 1e-16))[:, None]
    lv2_ref[...] = (sev2 / cmax)[:, None]


def _node_scalars(hs, hd, cnt_p, sev1_p, sev2_p, den_p):
    return pl.pallas_call(
        _k4a_body,
        out_shape=[
            jax.ShapeDtypeStruct((NP, 1), jnp.float32),
            jax.ShapeDtypeStruct((NP, 1), jnp.float32),
            jax.ShapeDtypeStruct((NP, 1), jnp.float32),
        ],
    )(hs, hd, cnt_p, sev1_p, sev2_p, den_p)


def _k4b_body(outp_ref, h_ref, p1l_ref, inv1_ref, lv2_ref, w2_ref, sc2_ref,
              b1_ref, g_ref, gs_ref, gd_ref, p2l_ref, p2lg_ref):
    op = outp_ref[...]
    p1l = p1l_ref[...]
    inv1 = inv1_ref[...]
    out1 = (op[0] + op[1] + p1l * h_ref[...]) * inv1 + b1_ref[...]
    h1 = jnp.where(out1 > 0, out1, jnp.exp(out1) - 1.0)
    g = jnp.dot(h1, w2_ref[...], preferred_element_type=jnp.float32)
    as2 = sc2_ref[0, 0]
    ad2 = sc2_ref[0, 1]
    g_ref[...] = g
    gs_ref[...] = as2 * g
    gd_ref[...] = ad2 * g
    a2 = (as2 + ad2) * g + lv2_ref[...]
    a2 = jnp.maximum(a2, 0.2 * a2)
    p2l = jnp.exp(a2)
    p2l_ref[...] = p2l
    p2lg_ref[...] = p2l * g


def _combine1(outp, h, p1l, inv1, lv2, W2, sc2, b1):
    B = 2048
    return pl.pallas_call(
        _k4b_body,
        grid=(NP // B,),
        in_specs=[
            pl.BlockSpec((NC, B, H), lambda i: (0, i, 0)),
            pl.BlockSpec((B, H), lambda i: (i, 0)),
            pl.BlockSpec((B, 1), lambda i: (i, 0)),
            pl.BlockSpec((B, 1), lambda i: (i, 0)),
            pl.BlockSpec((B, 1), lambda i: (i, 0)),
            pl.BlockSpec((H, 1), lambda i: (0, 0)),
            pl.BlockSpec(memory_space=pltpu.SMEM),
            pl.BlockSpec((1, H), lambda i: (0, 0)),
        ],
        out_specs=[pl.BlockSpec((B, 1), lambda i: (i, 0))] * 5,
        out_shape=[jax.ShapeDtypeStruct((NP, 1), jnp.float32)] * 5,
    )(outp, h, p1l, inv1, lv2, W2, sc2, b1)


def _k6_body(den2_ref, num2_ref, p2l_ref, p2lg_ref, b2_ref, out_ref):
    den = den2_ref[...][0] + den2_ref[...][1] + p2l_ref[...][:, 0]
    num = num2_ref[...][0] + num2_ref[...][1] + p2lg_ref[...][:, 0]
    out_ref[...] = (num / (den + 1e-16))[:, None] + b2_ref[0, 0]


def _combine2(den2_p, num2_p, p2l, p2lg, b2):
    return pl.pallas_call(
        _k6_body,
        in_specs=[
            pl.BlockSpec(),
            pl.BlockSpec(),
            pl.BlockSpec(),
            pl.BlockSpec(),
            pl.BlockSpec(memory_space=pltpu.SMEM),
        ],
        out_shape=jax.ShapeDtypeStruct((NP, 1), jnp.float32),
    )(den2_p, num2_p, p2l, p2lg, b2)


def kernel(x, edge_index, edge_attr, W1, a_src1, a_dst1, We1, a_edge1, b1,
           W2, a_src2, a_dst2, We2, a_edge2, b2):
    src = edge_index[0].astype(jnp.int32)
    dst = edge_index[1].astype(jnp.int32)

    xp = jnp.pad(x, ((0, NP - N), (0, 0)))
    h, hs, hd = _node_matmuls(xp, W1, a_src1, a_dst1)
    eav = _edge_scalars(edge_attr, We1, a_edge1, We2, a_edge2)
    eav1 = eav[:, 0] + 0.0
    eav2 = eav[:, 1] + 0.0

    outp, cnt_f, sev1_f, sev2_f, den_f = _get_edge_pass1()(
        src, dst, eav1, eav2, hs.reshape(NP), hd.reshape(NP), h)

    p1l, inv1, lv2 = _node_scalars(hs, hd,
                                   cnt_f.reshape(NC, NP),
                                   sev1_f.reshape(NC, NP),
                                   sev2_f.reshape(NC, NP),
                                   den_f.reshape(NC, NP))

    sc2 = jnp.stack([a_src2[0], a_dst2[0]]).reshape(1, 2)
    g, gs, gd, p2l, p2lg = _combine1(outp, h, p1l, inv1, lv2,
                                     W2, sc2, b1.reshape(1, H))

    den2_f, num2_f = _get_edge_pass2()(src, dst, eav2, g.reshape(NP),
                                       gs.reshape(NP), gd.reshape(NP))

    out = _combine2(den2_f.reshape(NC, NP), num2_f.reshape(NC, NP),
                    p2l, p2lg, b2.reshape(1, 1))
    return out[:N]
